# baseline (speedup 1.0000x reference)
import functools

import jax
import jax.numpy as jnp
from jax import lax
from jax.experimental import pallas as pl
from jax.experimental.pallas import tpu as pltpu
from jax.experimental.pallas import tpu_sc as plsc

_CH = 8
_V = 256


def _seg_max_sc(section_scores, b, s, c, vid_base, nvid):
    nw = 32
    vpw = nvid // nw
    nchunks = vpw // _CH
    rows_per_chunk = _CH * s

    mesh = plsc.VectorSubcoreMesh(core_axis_name="c", subcore_axis_name="s")

    @functools.partial(
        pl.kernel,
        mesh=mesh,
        out_type=jax.ShapeDtypeStruct((nvid, c), jnp.float32),
        scratch_types=[
            pltpu.VMEM((rows_per_chunk, c), jnp.float32),
            pltpu.VMEM((rows_per_chunk, c), jnp.float32),
            pltpu.VMEM((_CH, c), jnp.float32),
            pltpu.VMEM((_CH, c), jnp.float32),
            pltpu.SemaphoreType.DMA,
            pltpu.SemaphoreType.DMA,
            pltpu.SemaphoreType.DMA,
            pltpu.SemaphoreType.DMA,
        ],
        compiler_params=pltpu.CompilerParams(use_tc_tiling_on_sc=True),
    )
    def k(x_hbm, out_hbm, b0, b1, o0, o1, si0, si1, so0, so1):
        wid = lax.axis_index("s") * 2 + lax.axis_index("c")
        vid0 = wid * vpw

        def in_desc(ci, buf, sem):
            row0 = (vid_base + vid0 + ci * _CH) * s
            return pltpu.make_async_copy(
                x_hbm.at[pl.ds(row0, rows_per_chunk)], buf, sem)

        def out_desc(ci, ob, sem):
            return pltpu.make_async_copy(
                ob, out_hbm.at[pl.ds(vid0 + ci * _CH, _CH)], sem)

        def compute(buf, ob):
            def video_body(v, _):
                for g in range(c // 16):
                    sl = pl.ds(g * 16, 16)
                    acc = buf[v * s, sl]
                    for r in range(1, s):
                        acc = jnp.maximum(acc, buf[v * s + r, sl])
                    ob[v, sl] = acc
                return 0

            lax.fori_loop(0, _CH, video_body, 0, unroll=2)

        in_desc(0, b0, si0).start()
        in_desc(1, b1, si1).start()

        def pair_body(k_, _):
            c0 = 2 * k_
            in_desc(c0, b0, si0).wait()

            @pl.when(k_ > 0)
            def _():
                out_desc(c0 - 2, o0, so0).wait()

            compute(b0, o0)
            out_desc(c0, o0, so0).start()

            @pl.when(k_ < nchunks // 2 - 1)
            def _():
                in_desc(c0 + 2, b0, si0).start()

            in_desc(c0 + 1, b1, si1).wait()

            @pl.when(k_ > 0)
            def _():
                out_desc(c0 - 1, o1, so1).wait()

            compute(b1, o1)
            out_desc(c0 + 1, o1, so1).start()

            @pl.when(k_ < nchunks // 2 - 1)
            def _():
                in_desc(c0 + 3, b1, si1).start()

            return 0

        lax.fori_loop(0, nchunks // 2, pair_body, 0)
        out_desc(nchunks - 2, o0, so0).wait()
        out_desc(nchunks - 1, o1, so1).wait()

    return k(section_scores)


def _bce_body(x_ref, v_ref, y_ref, out_ref):
    i = pl.program_id(0)
    pooled = x_ref[...]
    y = y_ref[...]
    v = v_ref[...]

    def bce_sum(p):
        logp = jnp.maximum(jnp.log(p), -100.0)
        log1mp = jnp.maximum(jnp.log1p(-p), -100.0)
        return jnp.sum(y * logp + (1.0 - y) * log1mp)

    s = bce_sum(v) + bce_sum(pooled)

    @pl.when(i == 0)
    def _():
        out_ref[0, 0] = 0.0

    out_ref[0, 0] += s


def _fused_body(s, x_ref, v_ref, y_ref, out_ref):
    i = pl.program_id(0)
    x = x_ref[...]
    pooled = jnp.max(x.reshape(_V, s, x.shape[-1]), axis=1)
    y = y_ref[...]
    v = v_ref[...]

    def bce_sum(p):
        logp = jnp.maximum(jnp.log(p), -100.0)
        log1mp = jnp.maximum(jnp.log1p(-p), -100.0)
        return jnp.sum(y * logp + (1.0 - y) * log1mp)

    sm = bce_sum(v) + bce_sum(pooled)

    @pl.when(i == 0)
    def _():
        out_ref[0, 0] = 0.0

    out_ref[0, 0] += sm


def _fused_tc_half(section_scores, video_scores, labels, s, c, nblk):
    return pl.pallas_call(
        functools.partial(_fused_body, s),
        grid=(nblk,),
        in_specs=[
            pl.BlockSpec((_V * s, c), lambda i: (i, 0)),
            pl.BlockSpec((_V, c), lambda i: (i, 0)),
            pl.BlockSpec((_V, c), lambda i: (i, 0)),
        ],
        out_specs=pl.BlockSpec((1, 1), lambda i: (0, 0), memory_space=pltpu.SMEM),
        out_shape=jax.ShapeDtypeStruct((1, 1), jnp.float32),
    )(section_scores, video_scores, labels)


def _bce_pair_body(x_ref, y_ref, out_ref):
    i = pl.program_id(0)
    p = x_ref[...]
    y = y_ref[...]
    logp = jnp.maximum(jnp.log(p), -100.0)
    log1mp = jnp.maximum(jnp.log1p(-p), -100.0)
    sm = jnp.sum(y * logp + (1.0 - y) * log1mp)

    @pl.when(i == 0)
    def _():
        out_ref[0, 0] = 0.0

    out_ref[0, 0] += sm


def _bce_pair(scores, labels, c, score_off, lbl_off, nblk):
    return pl.pallas_call(
        _bce_pair_body,
        grid=(nblk,),
        in_specs=[
            pl.BlockSpec((_V, c), lambda i: (i + score_off, 0)),
            pl.BlockSpec((_V, c), lambda i: (i + lbl_off, 0)),
        ],
        out_specs=pl.BlockSpec((1, 1), lambda i: (0, 0), memory_space=pltpu.SMEM),
        out_shape=jax.ShapeDtypeStruct((1, 1), jnp.float32),
    )(scores, labels)


@jax.jit
def kernel(section_scores, video_scores, labels, segments):
    b, s = segments.shape
    c = section_scores.shape[1]
    h = (b * 6) // 16
    nhi = (b - h) // _V
    pooled_hi = _seg_max_sc(section_scores, b, s, c, h, b - h)
    acc_lo = _fused_tc_half(section_scores, video_scores, labels, s, c, h // _V)
    acc_vhi = _bce_pair(video_scores, labels, c, h // _V, h // _V, nhi)
    acc_phi = _bce_pair(pooled_hi, labels, c, 0, h // _V, nhi)
    return -(acc_lo[0, 0] + acc_vhi[0, 0] + acc_phi[0, 0]) / (b * c)


# speedup vs baseline: 1.0207x; 1.0207x over previous
import functools

import jax
import jax.numpy as jnp
from jax import lax
from jax.experimental import pallas as pl
from jax.experimental.pallas import tpu as pltpu
from jax.experimental.pallas import tpu_sc as plsc

_CH = 8
_V = 256


def _seg_max_sc(section_scores, b, s, c, vid_base, nvid):
    nw = 32
    vpw = nvid // nw
    nchunks = vpw // _CH
    rows_per_chunk = _CH * s

    mesh = plsc.VectorSubcoreMesh(core_axis_name="c", subcore_axis_name="s")

    @functools.partial(
        pl.kernel,
        mesh=mesh,
        out_type=jax.ShapeDtypeStruct((nvid, c), jnp.float32),
        scratch_types=[
            pltpu.VMEM((rows_per_chunk, c), jnp.float32),
            pltpu.VMEM((rows_per_chunk, c), jnp.float32),
            pltpu.VMEM((_CH, c), jnp.float32),
            pltpu.VMEM((_CH, c), jnp.float32),
            pltpu.SemaphoreType.DMA,
            pltpu.SemaphoreType.DMA,
            pltpu.SemaphoreType.DMA,
            pltpu.SemaphoreType.DMA,
        ],
        compiler_params=pltpu.CompilerParams(use_tc_tiling_on_sc=True),
    )
    def k(x_hbm, out_hbm, b0, b1, o0, o1, si0, si1, so0, so1):
        wid = lax.axis_index("s") * 2 + lax.axis_index("c")
        vid0 = wid * vpw

        def in_desc(ci, buf, sem):
            row0 = (vid_base + vid0 + ci * _CH) * s
            return pltpu.make_async_copy(
                x_hbm.at[pl.ds(row0, rows_per_chunk)], buf, sem)

        def out_desc(ci, ob, sem):
            return pltpu.make_async_copy(
                ob, out_hbm.at[pl.ds(vid0 + ci * _CH, _CH)], sem)

        def compute(buf, ob):
            def video_body(v, _):
                for g in range(c // 16):
                    sl = pl.ds(g * 16, 16)
                    acc = buf[v * s, sl]
                    for r in range(1, s):
                        acc = jnp.maximum(acc, buf[v * s + r, sl])
                    ob[v, sl] = acc
                return 0

            lax.fori_loop(0, _CH, video_body, 0, unroll=2)

        in_desc(0, b0, si0).start()
        in_desc(1, b1, si1).start()

        def pair_body(k_, _):
            c0 = 2 * k_
            in_desc(c0, b0, si0).wait()

            @pl.when(k_ > 0)
            def _():
                out_desc(c0 - 2, o0, so0).wait()

            compute(b0, o0)
            out_desc(c0, o0, so0).start()

            @pl.when(k_ < nchunks // 2 - 1)
            def _():
                in_desc(c0 + 2, b0, si0).start()

            in_desc(c0 + 1, b1, si1).wait()

            @pl.when(k_ > 0)
            def _():
                out_desc(c0 - 1, o1, so1).wait()

            compute(b1, o1)
            out_desc(c0 + 1, o1, so1).start()

            @pl.when(k_ < nchunks // 2 - 1)
            def _():
                in_desc(c0 + 3, b1, si1).start()

            return 0

        lax.fori_loop(0, nchunks // 2, pair_body, 0)
        out_desc(nchunks - 2, o0, so0).wait()
        out_desc(nchunks - 1, o1, so1).wait()

    return k(section_scores)


def _bce_body(x_ref, v_ref, y_ref, out_ref):
    i = pl.program_id(0)
    pooled = x_ref[...]
    y = y_ref[...]
    v = v_ref[...]

    def bce_sum(p):
        logp = jnp.maximum(jnp.log(p), -100.0)
        log1mp = jnp.maximum(jnp.log1p(-p), -100.0)
        return jnp.sum(y * logp + (1.0 - y) * log1mp)

    s = bce_sum(v) + bce_sum(pooled)

    @pl.when(i == 0)
    def _():
        out_ref[0, 0] = 0.0

    out_ref[0, 0] += s


def _fused_body(s, x_ref, v_ref, y_ref, out_ref):
    i = pl.program_id(0)
    x = x_ref[...]
    pooled = jnp.max(x.reshape(_V, s, x.shape[-1]), axis=1)
    y = y_ref[...]
    v = v_ref[...]

    def bce_sum(p):
        logp = jnp.maximum(jnp.log(p), -100.0)
        log1mp = jnp.maximum(jnp.log1p(-p), -100.0)
        return jnp.sum(y * logp + (1.0 - y) * log1mp)

    sm = bce_sum(v) + bce_sum(pooled)

    @pl.when(i == 0)
    def _():
        out_ref[0, 0] = 0.0

    out_ref[0, 0] += sm


def _fused_tc_half(section_scores, video_scores, labels, s, c, nblk):
    return pl.pallas_call(
        functools.partial(_fused_body, s),
        grid=(nblk,),
        in_specs=[
            pl.BlockSpec((_V * s, c), lambda i: (i, 0)),
            pl.BlockSpec((_V, c), lambda i: (i, 0)),
            pl.BlockSpec((_V, c), lambda i: (i, 0)),
        ],
        out_specs=pl.BlockSpec((1, 1), lambda i: (0, 0), memory_space=pltpu.SMEM),
        out_shape=jax.ShapeDtypeStruct((1, 1), jnp.float32),
    )(section_scores, video_scores, labels)


@jax.jit
def kernel(section_scores, video_scores, labels, segments):
    b, s = segments.shape
    c = section_scores.shape[1]
    h = (b * 7) // 16
    pooled_hi = _seg_max_sc(section_scores, b, s, c, h, b - h)
    acc_lo = _fused_tc_half(section_scores, video_scores, labels, s, c, h // _V)
    acc_hi = pl.pallas_call(
        _bce_body,
        grid=((b - h) // _V,),
        in_specs=[
            pl.BlockSpec((_V, c), lambda i: (i, 0)),
            pl.BlockSpec((_V, c), lambda i: (i + (b * 7) // 16 // _V, 0)),
            pl.BlockSpec((_V, c), lambda i: (i + (b * 7) // 16 // _V, 0)),
        ],
        out_specs=pl.BlockSpec((1, 1), lambda i: (0, 0), memory_space=pltpu.SMEM),
        out_shape=jax.ShapeDtypeStruct((1, 1), jnp.float32),
    )(pooled_hi, video_scores, labels)
    return -(acc_lo[0, 0] + acc_hi[0, 0]) / (b * c)
